# baseline (speedup 1.0000x reference)
import functools

import jax
import jax.numpy as jnp
from jax import lax
from jax.experimental import pallas as pl
from jax.experimental.pallas import tpu as pltpu
from jax.experimental.pallas import tpu_sc as plsc

N = 10000
E = 320000
D = 128
DK = 32
GEOM = 9
RH = 64

NC = 2
NS = 16
L = 16
NW = NC * NS
EPW = E // NW
C = 40
CPW = EPW // C
NB = 10
HB = CPW // 2
INV_SQRT_DK = 1.0 / (DK ** 0.5)


_BE = 3200


def _edge_dense_body(eg_ref, dist_ref, wkg_ref, wvg_ref, w1k_ref, b1k_ref,
                     w2k_ref, b2k_ref, w1v_ref, b1v_ref, w2v_ref, b2v_ref,
                     gk_ref, gv_ref):
    eg = eg_ref[...]
    dist = dist_ref[...]
    hk = jnp.maximum(dist * w1k_ref[...] + b1k_ref[...], 0.0)
    rk = jnp.dot(hk, w2k_ref[...], preferred_element_type=jnp.float32) + b2k_ref[...]
    hv = jnp.maximum(dist * w1v_ref[...] + b1v_ref[...], 0.0)
    rv = jnp.dot(hv, w2v_ref[...], preferred_element_type=jnp.float32) + b2v_ref[...]
    gk_ref[...] = jnp.dot(eg, wkg_ref[...], preferred_element_type=jnp.float32) * rk
    gv_ref[...] = jnp.dot(eg, wvg_ref[...], preferred_element_type=jnp.float32) * rv


def _edge_dense(eg, dist, p):
    def full(shape):
        return pl.BlockSpec(shape, lambda i: (0, 0))
    return pl.pallas_call(
        _edge_dense_body,
        grid=(E // _BE,),
        in_specs=[
            pl.BlockSpec((_BE, GEOM), lambda i: (i, 0)),
            pl.BlockSpec((_BE, 1), lambda i: (i, 0)),
            full((GEOM, DK)), full((GEOM, D)),
            full((1, RH)), full((1, RH)),
            full((RH, DK)), full((1, DK)),
            full((1, RH)), full((1, RH)),
            full((RH, D)), full((1, D)),
        ],
        out_specs=[
            pl.BlockSpec((_BE, DK), lambda i: (i, 0)),
            pl.BlockSpec((_BE, D), lambda i: (i, 0)),
        ],
        out_shape=[
            jax.ShapeDtypeStruct((E, DK), jnp.float32),
            jax.ShapeDtypeStruct((E, D), jnp.float32),
        ],
    )(eg, dist, p["Wkg"], p["Wvg"],
      p["W1k"], p["b1k"].reshape(1, RH), p["W2k"], p["b2k"].reshape(1, DK),
      p["W1v"], p["b1v"].reshape(1, RH), p["W2v"], p["b2v"].reshape(1, D))


def _proj(x, wq_ref, wkf_ref, wvf_ref, wvs_ref, q_ref, kf_ref, vf_ref, vs_ref):
    q_ref[...] = jnp.dot(x, wq_ref[...], preferred_element_type=jnp.float32)
    kf_ref[...] = jnp.dot(x, wkf_ref[...], preferred_element_type=jnp.float32)
    vf_ref[...] = jnp.dot(x, wvf_ref[...], preferred_element_type=jnp.float32)
    vs_ref[...] = jnp.dot(x, wvs_ref[...], preferred_element_type=jnp.float32)


def _node_dense_body(x_ref, wq_ref, wkf_ref, wvf_ref, wvs_ref,
                     q_ref, kf_ref, vf_ref, vs_ref):
    _proj(x_ref[...], wq_ref, wkf_ref, wvf_ref, wvs_ref,
          q_ref, kf_ref, vf_ref, vs_ref)


_NODE_OUT = [
    jax.ShapeDtypeStruct((N, D), jnp.float32),
    jax.ShapeDtypeStruct((N, D), jnp.float32),
    jax.ShapeDtypeStruct((N, D), jnp.float32),
    jax.ShapeDtypeStruct((N, D), jnp.float32),
]


def _pad_w(w):
    return jnp.pad(w, ((0, 0), (0, D - w.shape[1])))


def _node_dense(x, p):
    return pl.pallas_call(_node_dense_body, out_shape=_NODE_OUT)(
        x, _pad_w(p["Wq"]), _pad_w(p["Wkf"]), p["Wvf"], p["Wvs"])


def _combine(num_ref, den_ref, vs_ref):
    num = num_ref[0] + num_ref[1] + vs_ref[...]
    den = 1.0 + den_ref[0, :N] + den_ref[1, :N]
    return num / den[:, None]


def _combine_node_body(num_ref, den_ref, vsp_ref, wq_ref, wkf_ref, wvf_ref,
                       wvs_ref, q_ref, kf_ref, vf_ref, vs_ref):
    x = _combine(num_ref, den_ref, vsp_ref)
    _proj(x, wq_ref, wkf_ref, wvf_ref, wvs_ref, q_ref, kf_ref, vf_ref, vs_ref)


def _combine_node(num_p, den_p, vs_prev, p):
    return pl.pallas_call(_combine_node_body, out_shape=_NODE_OUT)(
        num_p, den_p, vs_prev, _pad_w(p["Wq"]), _pad_w(p["Wkf"]),
        p["Wvf"], p["Wvs"])


def _combine_only_body(num_ref, den_ref, vsp_ref, x_ref):
    x_ref[...] = _combine(num_ref, den_ref, vsp_ref)


def _combine_only(num_p, den_p, vs_prev):
    return pl.pallas_call(
        _combine_only_body,
        out_shape=jax.ShapeDtypeStruct((N, D), jnp.float32),
    )(num_p, den_p, vs_prev)


_MESH = plsc.VectorSubcoreMesh(
    core_axis_name="c", subcore_axis_name="s", num_cores=NC, num_subcores=NS)

NDEN = 10112
_DEN0 = 7 * C * DK
_DENR = NDEN - _DEN0


@functools.partial(
    pl.kernel,
    out_type=[
        jax.ShapeDtypeStruct((NC, N, D), jnp.float32),
        jax.ShapeDtypeStruct((NC, NDEN), jnp.float32),
    ],
    mesh=_MESH,
    compiler_params=pltpu.CompilerParams(needs_layout_passes=False),
    scratch_types=[
        pltpu.VMEM((NB * C,), jnp.int32),
        pltpu.VMEM((NB * C,), jnp.int32),
        pltpu.VMEM((C, D), jnp.float32),
        pltpu.VMEM((C, D), jnp.float32),
        pltpu.VMEM((C, D), jnp.float32),
        pltpu.VMEM((C * DK,), jnp.float32),
        pltpu.VMEM((C, D), jnp.float32),
        pltpu.VMEM((C,), jnp.float32),
        pltpu.VMEM((C,), jnp.int32),
        pltpu.VMEM((C,), jnp.int32),
        pltpu.VMEM((C,), jnp.int32),
        pltpu.VMEM((C, D), jnp.float32),
        pltpu.VMEM((C, D), jnp.float32),
        pltpu.VMEM((C, D), jnp.float32),
        pltpu.VMEM((C * DK,), jnp.float32),
        pltpu.VMEM((C, D), jnp.float32),
        pltpu.VMEM((C,), jnp.float32),
        pltpu.VMEM((C,), jnp.int32),
        pltpu.VMEM((C,), jnp.int32),
        pltpu.VMEM((C,), jnp.int32),
        pltpu.VMEM_SHARED((N, D), jnp.float32),
        pltpu.VMEM_SHARED((NDEN,), jnp.float32),
        pltpu.SemaphoreType.DMA,
        pltpu.SemaphoreType.DMA,
        pltpu.SemaphoreType.DMA,
        pltpu.SemaphoreType.DMA,
    ],
)
def _sc_edge(src_hbm, dst_hbm, q_hbm, kf_hbm, vf_hbm, gk_hbm, gv_hbm,
             num_out, den_out,
             srcb, dstb,
             q0, kf0, vf0, gk0, gv0, ev0, sc0, dc0, ds0,
             q1, kf1, vf1, gk1, gv1, ev1, sc1, dc1, ds1,
             num_sh, den_sh, dsem0, dsem1, ssem0, ssem1):
    cid = lax.axis_index("c")
    sid = lax.axis_index("s")
    wid = sid * NC + cid
    cbase = wid * CPW
    z16 = jnp.zeros((L,), jnp.float32)
    sets = [(q0, kf0, vf0, gk0, gv0, ev0, sc0, dc0, ds0, dsem0, ssem0),
            (q1, kf1, vf1, gk1, gv1, ev1, sc1, dc1, ds1, dsem1, ssem1)]

    def _zvf(i, carry):
        for d in range(D // L):
            vf0[i, pl.ds(d * L, L)] = z16
        return carry
    lax.fori_loop(0, C, _zvf, 0)

    def _zgk(i, carry):
        gk0[pl.ds(i * L, L)] = z16
        return carry
    lax.fori_loop(0, C * DK // L, _zgk, 0)

    for k in range(CPW // NS):
        pltpu.sync_copy(vf0, num_sh.at[pl.ds((sid + k * NS) * C, C)])
    @pl.when(sid < CPW - (CPW // NS) * NS)
    def _():
        pltpu.sync_copy(vf0, num_sh.at[pl.ds((sid + (CPW // NS) * NS) * C, C)])
    @pl.when(sid < 7)
    def _():
        pltpu.sync_copy(gk0, den_sh.at[pl.ds(sid * C * DK, C * DK)])
    @pl.when(sid == 7)
    def _():
        pltpu.sync_copy(gk0.at[pl.ds(0, _DENR)], den_sh.at[pl.ds(_DEN0, _DENR)])
    plsc.subcore_barrier()

    def _fetch_idx(batch):
        off = (cbase + batch * NB) * C
        pltpu.sync_copy(src_hbm.at[pl.ds(off, NB * C)], srcb)
        pltpu.sync_copy(dst_hbm.at[pl.ds(off, NB * C)], dstb)

    def _start_main(ci, s):
        (qr, kfr, vfr, gkr, gvr, _, scc, dcc, _, dsem, _) = sets[s]
        gc = cbase + ci
        base = lax.rem(ci, NB) * C
        for off in (0, 16, 24):
            scc[pl.ds(off, L)] = srcb[pl.ds(base + off, L)]
            dcc[pl.ds(off, L)] = dstb[pl.ds(base + off, L)]
        pltpu.async_copy(q_hbm.at[dcc], qr, dsem)
        pltpu.async_copy(kf_hbm.at[scc], kfr, dsem)
        pltpu.async_copy(gk_hbm.at[gc], gkr, dsem)
        pltpu.async_copy(gv_hbm.at[pl.ds(gc * C, C)], gvr, dsem)

    def _start_vf(s):
        (_, _, vfr, _, _, _, scc, _, _, dsem, _) = sets[s]
        pltpu.async_copy(vf_hbm.at[scc], vfr, dsem)

    def _wait_inputs(s):
        (qr, kfr, vfr, gkr, gvr, _, scc, dcc, _, dsem, _) = sets[s]
        pltpu.make_async_copy(q_hbm.at[dcc], qr, dsem).wait()
        pltpu.make_async_copy(kf_hbm.at[scc], kfr, dsem).wait()
        pltpu.make_async_copy(vf_hbm.at[scc], vfr, dsem).wait()
        pltpu.make_async_copy(gk_hbm.at[0], gkr, dsem).wait()
        pltpu.make_async_copy(gv_hbm.at[pl.ds(0, C)], gvr, dsem).wait()

    def _wait_scatters(s):
        (_, _, vfr, _, _, evr, _, _, dss, _, ssem) = sets[s]
        pltpu.make_async_copy(vfr, num_sh.at[dss], ssem).wait()
        pltpu.make_async_copy(evr, den_sh.at[dss], ssem).wait()

    def _compute_and_scatter(s):
        (qr, kfr, vfr, gkr, gvr, evr, _, dcc, dss, _, ssem) = sets[s]
        for off in (0, 16, 24):
            dss[pl.ds(off, L)] = dcc[pl.ds(off, L)]
        for g0 in (0, 16, 24):
            eidx = lax.iota(jnp.int32, L) + g0
            fbase = eidx * DK
            accs = [z16, z16, z16, z16]
            for j in range(DK):
                jidx = jnp.full((L,), j, jnp.int32)
                accs[j % 4] = accs[j % 4] + (
                    plsc.load_gather(qr, [eidx, jidx]) *
                    plsc.load_gather(kfr, [eidx, jidx]) *
                    plsc.load_gather(gkr, [fbase + j]))
            acc = (accs[0] + accs[1]) + (accs[2] + accs[3])
            evr[pl.ds(g0, L)] = jnp.exp(acc * INV_SQRT_DK - 1.0)
        def _vrow(i, carry):
            e0 = i * 4
            for u in range(4):
                e = e0 + u
                evv = plsc.load_gather(evr, [jnp.full((L,), e, jnp.int32)])
                for d in range(D // L):
                    sl = pl.ds(d * L, L)
                    vfr[e, sl] = evv * (vfr[e, sl] * gvr[e, sl])
            return carry
        lax.fori_loop(0, C // 4, _vrow, 0)
        pltpu.async_copy(vfr, num_sh.at[dss], ssem, add=True)
        pltpu.async_copy(evr, den_sh.at[dss], ssem, add=True)

    _fetch_idx(0)
    _start_main(0, 0)
    _start_vf(0)

    def _body(k, carry):
        a = 2 * k
        @pl.when(k > 0)
        def _():
            _wait_scatters(1)
        _start_main(a + 1, 1)
        _start_vf(1)
        _wait_inputs(0)
        _compute_and_scatter(0)
        @pl.when(k < HB - 1)
        def _():
            @pl.when(lax.rem(a + 2, NB) == 0)
            def _():
                _fetch_idx((a + 2) // NB)
            _start_main(a + 2, 0)
        _wait_inputs(1)
        _compute_and_scatter(1)
        @pl.when(k < HB - 1)
        def _():
            _wait_scatters(0)
            _start_vf(0)
        return carry

    lax.fori_loop(0, HB, _body, 0)
    _wait_scatters(0)
    _wait_scatters(1)

    plsc.subcore_barrier()
    for k in range(CPW // NS):
        r0 = (sid + k * NS) * C
        pltpu.sync_copy(num_sh.at[pl.ds(r0, C)], vf0)
        pltpu.sync_copy(vf0, num_out.at[cid, pl.ds(r0, C)])
    @pl.when(sid < CPW - (CPW // NS) * NS)
    def _():
        r0 = (sid + (CPW // NS) * NS) * C
        pltpu.sync_copy(num_sh.at[pl.ds(r0, C)], vf0)
        pltpu.sync_copy(vf0, num_out.at[cid, pl.ds(r0, C)])
    @pl.when(sid < 7)
    def _():
        pltpu.sync_copy(den_sh.at[pl.ds(sid * C * DK, C * DK)], gk0)
        pltpu.sync_copy(gk0, den_out.at[cid, pl.ds(sid * C * DK, C * DK)])
    @pl.when(sid == 7)
    def _():
        pltpu.sync_copy(den_sh.at[pl.ds(_DEN0, _DENR)], gk0.at[pl.ds(0, _DENR)])
        pltpu.sync_copy(gk0.at[pl.ds(0, _DENR)],
                        den_out.at[cid, pl.ds(_DEN0, _DENR)])


def kernel(edge_index, node_features, edge_features, distances, params):
    src = edge_index[0].astype(jnp.int32)
    dst = edge_index[1].astype(jnp.int32)
    p1, p2 = params

    gk1, gv1 = _edge_dense(edge_features, distances, p1)
    gk2, gv2 = _edge_dense(edge_features, distances, p2)
    gk1 = gk1.reshape(E // C, C * DK)
    gk2 = gk2.reshape(E // C, C * DK)

    q1, kf1, vf1, vs1 = _node_dense(node_features, p1)
    num1, den1 = _sc_edge(src, dst, q1, kf1, vf1, gk1, gv1)
    q2, kf2, vf2, vs2 = _combine_node(num1, den1, vs1, p2)
    num2, den2 = _sc_edge(src, dst, q2, kf2, vf2, gk2, gv2)
    return _combine_only(num2, den2, vs2)


# speedup vs baseline: 1.4757x; 1.4757x over previous
import functools

import jax
import jax.numpy as jnp
from jax import lax
from jax.experimental import pallas as pl
from jax.experimental.pallas import tpu as pltpu
from jax.experimental.pallas import tpu_sc as plsc

N = 10000
E = 320000
D = 128
DK = 32
GEOM = 9
RH = 64

NC = 2
NS = 16
L = 16
NW = NC * NS
EPW = E // NW
C = 40
CPW = EPW // C
NB = 10
HB = CPW // 2
INV_SQRT_DK = 1.0 / (DK ** 0.5)


_BE = 3200


def _edge_dense_body(eg_ref, dist_ref, wkg_ref, wvg_ref, w1k_ref, b1k_ref,
                     w2k_ref, b2k_ref, w1v_ref, b1v_ref, w2v_ref, b2v_ref,
                     gk_ref, gv_ref):
    eg = eg_ref[...]
    dist = dist_ref[...]
    hk = jnp.maximum(dist * w1k_ref[...] + b1k_ref[...], 0.0)
    rk = jnp.dot(hk, w2k_ref[...], preferred_element_type=jnp.float32) + b2k_ref[...]
    hv = jnp.maximum(dist * w1v_ref[...] + b1v_ref[...], 0.0)
    rv = jnp.dot(hv, w2v_ref[...], preferred_element_type=jnp.float32) + b2v_ref[...]
    gk_ref[...] = jnp.dot(eg, wkg_ref[...], preferred_element_type=jnp.float32) * rk
    gv_ref[...] = jnp.dot(eg, wvg_ref[...], preferred_element_type=jnp.float32) * rv


def _edge_dense(eg, dist, p):
    def full(shape):
        return pl.BlockSpec(shape, lambda i: (0, 0))
    return pl.pallas_call(
        _edge_dense_body,
        grid=(E // _BE,),
        in_specs=[
            pl.BlockSpec((_BE, GEOM), lambda i: (i, 0)),
            pl.BlockSpec((_BE, 1), lambda i: (i, 0)),
            full((GEOM, DK)), full((GEOM, D)),
            full((1, RH)), full((1, RH)),
            full((RH, DK)), full((1, DK)),
            full((1, RH)), full((1, RH)),
            full((RH, D)), full((1, D)),
        ],
        out_specs=[
            pl.BlockSpec((_BE, DK), lambda i: (i, 0)),
            pl.BlockSpec((_BE, D), lambda i: (i, 0)),
        ],
        out_shape=[
            jax.ShapeDtypeStruct((E, DK), jnp.float32),
            jax.ShapeDtypeStruct((E, D), jnp.float32),
        ],
    )(eg, dist, p["Wkg"], p["Wvg"],
      p["W1k"], p["b1k"].reshape(1, RH), p["W2k"], p["b2k"].reshape(1, DK),
      p["W1v"], p["b1v"].reshape(1, RH), p["W2v"], p["b2v"].reshape(1, D))


def _proj(x, wq_ref, wkf_ref, wvf_ref, wvs_ref, q_ref, kf_ref, vf_ref, vs_ref):
    q_ref[...] = jnp.dot(x, wq_ref[...], preferred_element_type=jnp.float32)
    kf_ref[...] = jnp.dot(x, wkf_ref[...], preferred_element_type=jnp.float32)
    vf_ref[...] = jnp.dot(x, wvf_ref[...], preferred_element_type=jnp.float32)
    vs_ref[...] = jnp.dot(x, wvs_ref[...], preferred_element_type=jnp.float32)


def _node_dense_body(x_ref, wq_ref, wkf_ref, wvf_ref, wvs_ref,
                     q_ref, kf_ref, vf_ref, vs_ref):
    _proj(x_ref[...], wq_ref, wkf_ref, wvf_ref, wvs_ref,
          q_ref, kf_ref, vf_ref, vs_ref)


_NODE_OUT = [
    jax.ShapeDtypeStruct((N, D), jnp.float32),
    jax.ShapeDtypeStruct((N, D), jnp.float32),
    jax.ShapeDtypeStruct((N, D), jnp.float32),
    jax.ShapeDtypeStruct((N, D), jnp.float32),
]


def _pad_w(w):
    return jnp.pad(w, ((0, 0), (0, D - w.shape[1])))


def _node_dense(x, p):
    return pl.pallas_call(_node_dense_body, out_shape=_NODE_OUT)(
        x, _pad_w(p["Wq"]), _pad_w(p["Wkf"]), p["Wvf"], p["Wvs"])


def _combine(num_ref, den_ref, vs_ref):
    num = num_ref[0] + num_ref[1] + vs_ref[...]
    den = 1.0 + den_ref[0, :N] + den_ref[1, :N]
    return num / den[:, None]


def _combine_node_body(num_ref, den_ref, vsp_ref, wq_ref, wkf_ref, wvf_ref,
                       wvs_ref, q_ref, kf_ref, vf_ref, vs_ref):
    x = _combine(num_ref, den_ref, vsp_ref)
    _proj(x, wq_ref, wkf_ref, wvf_ref, wvs_ref, q_ref, kf_ref, vf_ref, vs_ref)


def _combine_node(num_p, den_p, vs_prev, p):
    return pl.pallas_call(_combine_node_body, out_shape=_NODE_OUT)(
        num_p, den_p, vs_prev, _pad_w(p["Wq"]), _pad_w(p["Wkf"]),
        p["Wvf"], p["Wvs"])


def _combine_only_body(num_ref, den_ref, vsp_ref, x_ref):
    x_ref[...] = _combine(num_ref, den_ref, vsp_ref)


def _combine_only(num_p, den_p, vs_prev):
    return pl.pallas_call(
        _combine_only_body,
        out_shape=jax.ShapeDtypeStruct((N, D), jnp.float32),
    )(num_p, den_p, vs_prev)


_MESH = plsc.VectorSubcoreMesh(
    core_axis_name="c", subcore_axis_name="s", num_cores=NC, num_subcores=NS)

NDEN = 10112
_DEN0 = 7 * C * DK
_DENR = NDEN - _DEN0


@functools.partial(
    pl.kernel,
    out_type=[
        jax.ShapeDtypeStruct((NC, N, D), jnp.float32),
        jax.ShapeDtypeStruct((NC, NDEN), jnp.float32),
    ],
    mesh=_MESH,
    compiler_params=pltpu.CompilerParams(needs_layout_passes=False),
    scratch_types=[
        pltpu.VMEM((NB * C,), jnp.int32),
        pltpu.VMEM((NB * C,), jnp.int32),
        pltpu.VMEM((C, D), jnp.float32),
        pltpu.VMEM((C, D), jnp.float32),
        pltpu.VMEM((C, D), jnp.float32),
        pltpu.VMEM((C * DK,), jnp.float32),
        pltpu.VMEM((C, D), jnp.float32),
        pltpu.VMEM((C,), jnp.float32),
        pltpu.VMEM((C,), jnp.int32),
        pltpu.VMEM((C,), jnp.int32),
        pltpu.VMEM((C, D), jnp.float32),
        pltpu.VMEM((C, D), jnp.float32),
        pltpu.VMEM((C, D), jnp.float32),
        pltpu.VMEM((C * DK,), jnp.float32),
        pltpu.VMEM((C, D), jnp.float32),
        pltpu.VMEM((C,), jnp.float32),
        pltpu.VMEM((C,), jnp.int32),
        pltpu.VMEM((C,), jnp.int32),
        pltpu.VMEM_SHARED((N, D), jnp.float32),
        pltpu.VMEM_SHARED((NDEN,), jnp.float32),
        pltpu.SemaphoreType.DMA,
        pltpu.SemaphoreType.DMA,
        pltpu.SemaphoreType.DMA,
        pltpu.SemaphoreType.DMA,
    ],
)
def _sc_edge(src_hbm, dst_hbm, q_hbm, kf_hbm, vf_hbm, gk_hbm, gv_hbm,
             num_out, den_out,
             srcb, dstb,
             q0, kf0, vf0, gk0, gv0, ev0, sc0, dc0,
             q1, kf1, vf1, gk1, gv1, ev1, sc1, dc1,
             num_sh, den_sh, dsem0, dsem1, ssem0, ssem1):
    cid = lax.axis_index("c")
    sid = lax.axis_index("s")
    wid = sid * NC + cid
    cbase = wid * CPW
    z16 = jnp.zeros((L,), jnp.float32)
    sets = [(q0, kf0, vf0, gk0, gv0, ev0, sc0, dc0, dsem0, ssem0),
            (q1, kf1, vf1, gk1, gv1, ev1, sc1, dc1, dsem1, ssem1)]

    def _zvf(i, carry):
        for d in range(D // L):
            vf0[i, pl.ds(d * L, L)] = z16
        return carry
    lax.fori_loop(0, C, _zvf, 0)

    def _zgk(i, carry):
        gk0[pl.ds(i * L, L)] = z16
        return carry
    lax.fori_loop(0, C * DK // L, _zgk, 0)

    for k in range(CPW // NS):
        pltpu.sync_copy(vf0, num_sh.at[pl.ds((sid + k * NS) * C, C)])
    @pl.when(sid < CPW - (CPW // NS) * NS)
    def _():
        pltpu.sync_copy(vf0, num_sh.at[pl.ds((sid + (CPW // NS) * NS) * C, C)])
    @pl.when(sid < 7)
    def _():
        pltpu.sync_copy(gk0, den_sh.at[pl.ds(sid * C * DK, C * DK)])
    @pl.when(sid == 7)
    def _():
        pltpu.sync_copy(gk0.at[pl.ds(0, _DENR)], den_sh.at[pl.ds(_DEN0, _DENR)])
    plsc.subcore_barrier()

    def _fetch_idx(batch):
        off = (cbase + batch * NB) * C
        pltpu.sync_copy(src_hbm.at[pl.ds(off, NB * C)], srcb)
        pltpu.sync_copy(dst_hbm.at[pl.ds(off, NB * C)], dstb)

    def _start_inputs(ci, s):
        (qr, kfr, vfr, gkr, gvr, _, scc, dcc, dsem, _) = sets[s]
        gc = cbase + ci
        base = lax.rem(ci, NB) * C
        for off in (0, 16, 24):
            scc[pl.ds(off, L)] = srcb[pl.ds(base + off, L)]
            dcc[pl.ds(off, L)] = dstb[pl.ds(base + off, L)]
        pltpu.async_copy(q_hbm.at[dcc], qr, dsem)
        pltpu.async_copy(kf_hbm.at[scc], kfr, dsem)
        pltpu.async_copy(vf_hbm.at[scc], vfr, dsem)
        pltpu.async_copy(gk_hbm.at[gc], gkr, dsem)
        pltpu.async_copy(gv_hbm.at[pl.ds(gc * C, C)], gvr, dsem)

    def _wait_inputs(s):
        (qr, kfr, vfr, gkr, gvr, _, scc, dcc, dsem, _) = sets[s]
        pltpu.make_async_copy(q_hbm.at[dcc], qr, dsem).wait()
        pltpu.make_async_copy(kf_hbm.at[scc], kfr, dsem).wait()
        pltpu.make_async_copy(vf_hbm.at[scc], vfr, dsem).wait()
        pltpu.make_async_copy(gk_hbm.at[0], gkr, dsem).wait()
        pltpu.make_async_copy(gv_hbm.at[pl.ds(0, C)], gvr, dsem).wait()

    def _wait_scatters(s):
        (_, _, vfr, _, _, evr, _, dcc, _, ssem) = sets[s]
        pltpu.make_async_copy(vfr, num_sh.at[dcc], ssem).wait()
        pltpu.make_async_copy(evr, den_sh.at[dcc], ssem).wait()

    def _compute_and_scatter(s):
        (qr, kfr, vfr, gkr, gvr, evr, _, dcc, _, ssem) = sets[s]
        for g0 in (0, 16, 24):
            acc = z16
            eidx = lax.iota(jnp.int32, L) + g0
            fbase = eidx * DK
            for j in range(DK):
                jidx = jnp.full((L,), j, jnp.int32)
                acc = acc + (plsc.load_gather(qr, [eidx, jidx]) *
                             plsc.load_gather(kfr, [eidx, jidx]) *
                             plsc.load_gather(gkr, [fbase + j]))
            evr[pl.ds(g0, L)] = jnp.exp(acc * INV_SQRT_DK - 1.0)
        @plsc.parallel_loop(0, C, step=1, unroll=4)
        def _vrow(e):
            evv = plsc.load_gather(evr, [jnp.full((L,), e, jnp.int32)])
            for d in range(D // L):
                sl = pl.ds(d * L, L)
                vfr[e, sl] = evv * vfr[e, sl] * gvr[e, sl]
        pltpu.async_copy(vfr, num_sh.at[dcc], ssem, add=True)
        pltpu.async_copy(evr, den_sh.at[dcc], ssem, add=True)

    _fetch_idx(0)
    _start_inputs(0, 0)

    def _body(k, carry):
        a = 2 * k
        @pl.when(k > 0)
        def _():
            _wait_scatters(1)
        _start_inputs(a + 1, 1)
        _wait_inputs(0)
        _compute_and_scatter(0)
        _wait_inputs(1)
        @pl.when(k < HB - 1)
        def _():
            @pl.when(lax.rem(a + 2, NB) == 0)
            def _():
                _fetch_idx((a + 2) // NB)
            _wait_scatters(0)
            _start_inputs(a + 2, 0)
        _compute_and_scatter(1)
        return carry

    lax.fori_loop(0, HB, _body, 0)
    _wait_scatters(0)
    _wait_scatters(1)

    plsc.subcore_barrier()
    for k in range(CPW // NS):
        r0 = (sid + k * NS) * C
        pltpu.sync_copy(num_sh.at[pl.ds(r0, C)], vf0)
        pltpu.sync_copy(vf0, num_out.at[cid, pl.ds(r0, C)])
    @pl.when(sid < CPW - (CPW // NS) * NS)
    def _():
        r0 = (sid + (CPW // NS) * NS) * C
        pltpu.sync_copy(num_sh.at[pl.ds(r0, C)], vf0)
        pltpu.sync_copy(vf0, num_out.at[cid, pl.ds(r0, C)])
    @pl.when(sid < 7)
    def _():
        pltpu.sync_copy(den_sh.at[pl.ds(sid * C * DK, C * DK)], gk0)
        pltpu.sync_copy(gk0, den_out.at[cid, pl.ds(sid * C * DK, C * DK)])
    @pl.when(sid == 7)
    def _():
        pltpu.sync_copy(den_sh.at[pl.ds(_DEN0, _DENR)], gk0.at[pl.ds(0, _DENR)])
        pltpu.sync_copy(gk0.at[pl.ds(0, _DENR)],
                        den_out.at[cid, pl.ds(_DEN0, _DENR)])


def kernel(edge_index, node_features, edge_features, distances, params):
    src = edge_index[0].astype(jnp.int32)
    dst = edge_index[1].astype(jnp.int32)
    p1, p2 = params

    gk1, gv1 = _edge_dense(edge_features, distances, p1)
    gk2, gv2 = _edge_dense(edge_features, distances, p2)
    gk1 = gk1.reshape(E // C, C * DK)
    gk2 = gk2.reshape(E // C, C * DK)

    q1, kf1, vf1, vs1 = _node_dense(node_features, p1)
    num1, den1 = _sc_edge(src, dst, q1, kf1, vf1, gk1, gv1)
    q2, kf2, vf2, vs2 = _combine_node(num1, den1, vs1, p2)
    num2, den2 = _sc_edge(src, dst, q2, kf2, vf2, gk2, gv2)
    return _combine_only(num2, den2, vs2)


# speedup vs baseline: 1.5831x; 1.0728x over previous
import functools

import jax
import jax.numpy as jnp
from jax import lax
from jax.experimental import pallas as pl
from jax.experimental.pallas import tpu as pltpu
from jax.experimental.pallas import tpu_sc as plsc

N = 10000
E = 320000
D = 128
DK = 32
GEOM = 9
RH = 64

NC = 2
NS = 16
L = 16
NW = NC * NS
EPW = E // NW
C = 40
CPW = EPW // C
NB = 10
HB = CPW // 2
INV_SQRT_DK = 1.0 / (DK ** 0.5)


_BE = 3200


def _edge_dense_body(eg_ref, dist_ref, wkg_ref, wvg_ref, w1k_ref, b1k_ref,
                     w2k_ref, b2k_ref, w1v_ref, b1v_ref, w2v_ref, b2v_ref,
                     gk_ref, gv_ref):
    eg = eg_ref[...]
    dist = dist_ref[...]
    hk = jnp.maximum(dist * w1k_ref[...] + b1k_ref[...], 0.0)
    rk = jnp.dot(hk, w2k_ref[...], preferred_element_type=jnp.float32) + b2k_ref[...]
    hv = jnp.maximum(dist * w1v_ref[...] + b1v_ref[...], 0.0)
    rv = jnp.dot(hv, w2v_ref[...], preferred_element_type=jnp.float32) + b2v_ref[...]
    gk_ref[...] = jnp.dot(eg, wkg_ref[...], preferred_element_type=jnp.float32) * rk
    gv_ref[...] = jnp.dot(eg, wvg_ref[...], preferred_element_type=jnp.float32) * rv


def _edge_dense(eg, dist, p):
    def full(shape):
        return pl.BlockSpec(shape, lambda i: (0, 0))
    return pl.pallas_call(
        _edge_dense_body,
        grid=(E // _BE,),
        in_specs=[
            pl.BlockSpec((_BE, GEOM), lambda i: (i, 0)),
            pl.BlockSpec((_BE, 1), lambda i: (i, 0)),
            full((GEOM, DK)), full((GEOM, D)),
            full((1, RH)), full((1, RH)),
            full((RH, DK)), full((1, DK)),
            full((1, RH)), full((1, RH)),
            full((RH, D)), full((1, D)),
        ],
        out_specs=[
            pl.BlockSpec((_BE, DK), lambda i: (i, 0)),
            pl.BlockSpec((_BE, D), lambda i: (i, 0)),
        ],
        out_shape=[
            jax.ShapeDtypeStruct((E, DK), jnp.float32),
            jax.ShapeDtypeStruct((E, D), jnp.float32),
        ],
    )(eg, dist, p["Wkg"], p["Wvg"],
      p["W1k"], p["b1k"].reshape(1, RH), p["W2k"], p["b2k"].reshape(1, DK),
      p["W1v"], p["b1v"].reshape(1, RH), p["W2v"], p["b2v"].reshape(1, D))


def _proj(x, wq_ref, wkf_ref, wvf_ref, wvs_ref, q_ref, kf_ref, vf_ref, vs_ref):
    q_ref[...] = jnp.dot(x, wq_ref[...], preferred_element_type=jnp.float32)
    kf_ref[...] = jnp.dot(x, wkf_ref[...], preferred_element_type=jnp.float32)
    vf_ref[...] = jnp.dot(x, wvf_ref[...], preferred_element_type=jnp.float32)
    vs_ref[...] = jnp.dot(x, wvs_ref[...], preferred_element_type=jnp.float32)


def _node_dense_body(x_ref, wq_ref, wkf_ref, wvf_ref, wvs_ref,
                     q_ref, kf_ref, vf_ref, vs_ref):
    _proj(x_ref[...], wq_ref, wkf_ref, wvf_ref, wvs_ref,
          q_ref, kf_ref, vf_ref, vs_ref)


_NODE_OUT = [
    jax.ShapeDtypeStruct((N, D), jnp.float32),
    jax.ShapeDtypeStruct((N, D), jnp.float32),
    jax.ShapeDtypeStruct((N, D), jnp.float32),
    jax.ShapeDtypeStruct((N, D), jnp.float32),
]


def _pad_w(w):
    return jnp.pad(w, ((0, 0), (0, D - w.shape[1])))


def _node_dense(x, p):
    return pl.pallas_call(_node_dense_body, out_shape=_NODE_OUT)(
        x, _pad_w(p["Wq"]), _pad_w(p["Wkf"]), p["Wvf"], p["Wvs"])


def _combine(num_ref, den_ref, vs_ref):
    num = num_ref[0] + num_ref[1] + vs_ref[...]
    den = 1.0 + den_ref[0, :N] + den_ref[1, :N]
    return num / den[:, None]


def _combine_node_body(num_ref, den_ref, vsp_ref, wq_ref, wkf_ref, wvf_ref,
                       wvs_ref, q_ref, kf_ref, vf_ref, vs_ref):
    x = _combine(num_ref, den_ref, vsp_ref)
    _proj(x, wq_ref, wkf_ref, wvf_ref, wvs_ref, q_ref, kf_ref, vf_ref, vs_ref)


def _combine_node(num_p, den_p, vs_prev, p):
    return pl.pallas_call(_combine_node_body, out_shape=_NODE_OUT)(
        num_p, den_p, vs_prev, _pad_w(p["Wq"]), _pad_w(p["Wkf"]),
        p["Wvf"], p["Wvs"])


def _combine_only_body(num_ref, den_ref, vsp_ref, x_ref):
    x_ref[...] = _combine(num_ref, den_ref, vsp_ref)


def _combine_only(num_p, den_p, vs_prev):
    return pl.pallas_call(
        _combine_only_body,
        out_shape=jax.ShapeDtypeStruct((N, D), jnp.float32),
    )(num_p, den_p, vs_prev)


_MESH = plsc.VectorSubcoreMesh(
    core_axis_name="c", subcore_axis_name="s", num_cores=NC, num_subcores=NS)

NDEN = 10112
_DEN0 = 7 * C * DK
_DENR = NDEN - _DEN0


@functools.partial(
    pl.kernel,
    out_type=[
        jax.ShapeDtypeStruct((NC, N, D), jnp.float32),
        jax.ShapeDtypeStruct((NC, NDEN), jnp.float32),
    ],
    mesh=_MESH,
    compiler_params=pltpu.CompilerParams(needs_layout_passes=False),
    scratch_types=[
        pltpu.VMEM((NB * C,), jnp.int32),
        pltpu.VMEM((NB * C,), jnp.int32),
        pltpu.VMEM((C, D), jnp.float32),
        pltpu.VMEM((C, D), jnp.float32),
        pltpu.VMEM((C, D), jnp.float32),
        pltpu.VMEM((C * DK,), jnp.float32),
        pltpu.VMEM((C, D), jnp.float32),
        pltpu.VMEM((C,), jnp.float32),
        pltpu.VMEM((C,), jnp.int32),
        pltpu.VMEM((C,), jnp.int32),
        pltpu.VMEM((C, D), jnp.float32),
        pltpu.VMEM((C, D), jnp.float32),
        pltpu.VMEM((C, D), jnp.float32),
        pltpu.VMEM((C * DK,), jnp.float32),
        pltpu.VMEM((C, D), jnp.float32),
        pltpu.VMEM((C,), jnp.float32),
        pltpu.VMEM((C,), jnp.int32),
        pltpu.VMEM((C,), jnp.int32),
        pltpu.VMEM_SHARED((N, D), jnp.float32),
        pltpu.VMEM_SHARED((NDEN,), jnp.float32),
        pltpu.SemaphoreType.DMA,
        pltpu.SemaphoreType.DMA,
        pltpu.SemaphoreType.DMA,
        pltpu.SemaphoreType.DMA,
    ],
)
def _sc_edge(src_hbm, dst_hbm, q_hbm, kf_hbm, vf_hbm, gk_hbm, gv_hbm,
             num_out, den_out,
             srcb, dstb,
             q0, kf0, vf0, gk0, gv0, ev0, sc0, dc0,
             q1, kf1, vf1, gk1, gv1, ev1, sc1, dc1,
             num_sh, den_sh, dsem0, dsem1, ssem0, ssem1):
    cid = lax.axis_index("c")
    sid = lax.axis_index("s")
    wid = sid * NC + cid
    cbase = wid * CPW
    z16 = jnp.zeros((L,), jnp.float32)
    sets = [(q0, kf0, vf0, gk0, gv0, ev0, sc0, dc0, dsem0, ssem0),
            (q1, kf1, vf1, gk1, gv1, ev1, sc1, dc1, dsem1, ssem1)]

    def _zvf(i, carry):
        for d in range(D // L):
            vf0[i, pl.ds(d * L, L)] = z16
        return carry
    lax.fori_loop(0, C, _zvf, 0)

    def _zgk(i, carry):
        gk0[pl.ds(i * L, L)] = z16
        return carry
    lax.fori_loop(0, C * DK // L, _zgk, 0)

    for k in range(CPW // NS):
        pltpu.sync_copy(vf0, num_sh.at[pl.ds((sid + k * NS) * C, C)])
    @pl.when(sid < CPW - (CPW // NS) * NS)
    def _():
        pltpu.sync_copy(vf0, num_sh.at[pl.ds((sid + (CPW // NS) * NS) * C, C)])
    @pl.when(sid < 7)
    def _():
        pltpu.sync_copy(gk0, den_sh.at[pl.ds(sid * C * DK, C * DK)])
    @pl.when(sid == 7)
    def _():
        pltpu.sync_copy(gk0.at[pl.ds(0, _DENR)], den_sh.at[pl.ds(_DEN0, _DENR)])
    plsc.subcore_barrier()

    def _fetch_idx(batch):
        off = (cbase + batch * NB) * C
        pltpu.sync_copy(src_hbm.at[pl.ds(off, NB * C)], srcb)
        pltpu.sync_copy(dst_hbm.at[pl.ds(off, NB * C)], dstb)

    def _start_inputs(ci, s):
        (qr, kfr, vfr, gkr, gvr, _, scc, dcc, dsem, _) = sets[s]
        gc = cbase + ci
        base = lax.rem(ci, NB) * C
        for off in (0, 16, 24):
            scc[pl.ds(off, L)] = srcb[pl.ds(base + off, L)]
            dcc[pl.ds(off, L)] = dstb[pl.ds(base + off, L)]
        pltpu.async_copy(q_hbm.at[dcc], qr, dsem)
        pltpu.async_copy(kf_hbm.at[scc], kfr, dsem)
        pltpu.async_copy(vf_hbm.at[scc], vfr, dsem)
        pltpu.async_copy(gk_hbm.at[gc], gkr, dsem)
        pltpu.async_copy(gv_hbm.at[pl.ds(gc * C, C)], gvr, dsem)

    def _wait_inputs(s):
        (qr, kfr, vfr, gkr, gvr, _, scc, dcc, dsem, _) = sets[s]
        pltpu.make_async_copy(q_hbm.at[dcc], qr, dsem).wait()
        pltpu.make_async_copy(kf_hbm.at[scc], kfr, dsem).wait()
        pltpu.make_async_copy(vf_hbm.at[scc], vfr, dsem).wait()
        pltpu.make_async_copy(gk_hbm.at[0], gkr, dsem).wait()
        pltpu.make_async_copy(gv_hbm.at[pl.ds(0, C)], gvr, dsem).wait()

    def _wait_scatters(s):
        (_, _, vfr, _, _, evr, _, dcc, _, ssem) = sets[s]
        pltpu.make_async_copy(vfr, num_sh.at[dcc], ssem).wait()
        pltpu.make_async_copy(evr, den_sh.at[dcc], ssem).wait()

    def _compute_and_scatter(s):
        (qr, kfr, vfr, gkr, gvr, evr, _, dcc, _, ssem) = sets[s]
        for g0 in (0, 16, 24):
            acc = z16
            eidx = lax.iota(jnp.int32, L) + g0
            for j in range(DK):
                jidx = jnp.full((L,), j, jnp.int32)
                acc = acc + (plsc.load_gather(qr, [eidx, jidx]) *
                             plsc.load_gather(kfr, [eidx, jidx]) *
                             gkr[pl.ds(j * C + g0, L)])
            evr[pl.ds(g0, L)] = jnp.exp(acc * INV_SQRT_DK - 1.0)
        @plsc.parallel_loop(0, C, step=1, unroll=4)
        def _vrow(e):
            evv = plsc.load_gather(evr, [jnp.full((L,), e, jnp.int32)])
            for d in range(D // L):
                sl = pl.ds(d * L, L)
                vfr[e, sl] = evv * vfr[e, sl] * gvr[e, sl]
        pltpu.async_copy(vfr, num_sh.at[dcc], ssem, add=True)
        pltpu.async_copy(evr, den_sh.at[dcc], ssem, add=True)

    _fetch_idx(0)
    _start_inputs(0, 0)

    def _body(k, carry):
        a = 2 * k
        @pl.when(k > 0)
        def _():
            _wait_scatters(1)
        _start_inputs(a + 1, 1)
        _wait_inputs(0)
        _compute_and_scatter(0)
        _wait_inputs(1)
        @pl.when(k < HB - 1)
        def _():
            @pl.when(lax.rem(a + 2, NB) == 0)
            def _():
                _fetch_idx((a + 2) // NB)
            _wait_scatters(0)
            _start_inputs(a + 2, 0)
        _compute_and_scatter(1)
        return carry

    lax.fori_loop(0, HB, _body, 0)
    _wait_scatters(0)
    _wait_scatters(1)

    plsc.subcore_barrier()
    for k in range(CPW // NS):
        r0 = (sid + k * NS) * C
        pltpu.sync_copy(num_sh.at[pl.ds(r0, C)], vf0)
        pltpu.sync_copy(vf0, num_out.at[cid, pl.ds(r0, C)])
    @pl.when(sid < CPW - (CPW // NS) * NS)
    def _():
        r0 = (sid + (CPW // NS) * NS) * C
        pltpu.sync_copy(num_sh.at[pl.ds(r0, C)], vf0)
        pltpu.sync_copy(vf0, num_out.at[cid, pl.ds(r0, C)])
    @pl.when(sid < 7)
    def _():
        pltpu.sync_copy(den_sh.at[pl.ds(sid * C * DK, C * DK)], gk0)
        pltpu.sync_copy(gk0, den_out.at[cid, pl.ds(sid * C * DK, C * DK)])
    @pl.when(sid == 7)
    def _():
        pltpu.sync_copy(den_sh.at[pl.ds(_DEN0, _DENR)], gk0.at[pl.ds(0, _DENR)])
        pltpu.sync_copy(gk0.at[pl.ds(0, _DENR)],
                        den_out.at[cid, pl.ds(_DEN0, _DENR)])


def kernel(edge_index, node_features, edge_features, distances, params):
    src = edge_index[0].astype(jnp.int32)
    dst = edge_index[1].astype(jnp.int32)
    p1, p2 = params

    gk1, gv1 = _edge_dense(edge_features, distances, p1)
    gk2, gv2 = _edge_dense(edge_features, distances, p2)
    gk1 = gk1.reshape(E // C, C, DK).transpose(0, 2, 1).reshape(E // C, DK * C)
    gk2 = gk2.reshape(E // C, C, DK).transpose(0, 2, 1).reshape(E // C, DK * C)

    q1, kf1, vf1, vs1 = _node_dense(node_features, p1)
    num1, den1 = _sc_edge(src, dst, q1, kf1, vf1, gk1, gv1)
    q2, kf2, vf2, vs2 = _combine_node(num1, den1, vs1, p2)
    num2, den2 = _sc_edge(src, dst, q2, kf2, vf2, gk2, gv2)
    return _combine_only(num2, den2, vs2)


# speedup vs baseline: 1.6251x; 1.0266x over previous
import functools

import jax
import jax.numpy as jnp
from jax import lax
from jax.experimental import pallas as pl
from jax.experimental.pallas import tpu as pltpu
from jax.experimental.pallas import tpu_sc as plsc

N = 10000
E = 320000
D = 128
DK = 32
GEOM = 9
RH = 64

NC = 2
NS = 16
L = 16
NW = NC * NS
EPW = E // NW
C = 40
CPW = EPW // C
NB = 10
HB = CPW // 2
INV_SQRT_DK = 1.0 / (DK ** 0.5)


_BE = 3200


def _edge_dense_body(eg_ref, dist_ref, wkg_ref, wvg_ref, w1k_ref, b1k_ref,
                     w2k_ref, b2k_ref, w1v_ref, b1v_ref, w2v_ref, b2v_ref,
                     gk_ref, gv_ref):
    eg = eg_ref[...]
    dist = dist_ref[...]
    hk = jnp.maximum(dist * w1k_ref[...] + b1k_ref[...], 0.0)
    rk = jnp.dot(hk, w2k_ref[...], preferred_element_type=jnp.float32) + b2k_ref[...]
    hv = jnp.maximum(dist * w1v_ref[...] + b1v_ref[...], 0.0)
    rv = jnp.dot(hv, w2v_ref[...], preferred_element_type=jnp.float32) + b2v_ref[...]
    gk_ref[...] = jnp.dot(eg, wkg_ref[...], preferred_element_type=jnp.float32) * rk
    gv_ref[...] = jnp.dot(eg, wvg_ref[...], preferred_element_type=jnp.float32) * rv


def _edge_dense(eg, dist, p):
    def full(shape):
        return pl.BlockSpec(shape, lambda i: (0, 0))
    return pl.pallas_call(
        _edge_dense_body,
        grid=(E // _BE,),
        in_specs=[
            pl.BlockSpec((_BE, GEOM), lambda i: (i, 0)),
            pl.BlockSpec((_BE, 1), lambda i: (i, 0)),
            full((GEOM, DK)), full((GEOM, D)),
            full((1, RH)), full((1, RH)),
            full((RH, DK)), full((1, DK)),
            full((1, RH)), full((1, RH)),
            full((RH, D)), full((1, D)),
        ],
        out_specs=[
            pl.BlockSpec((_BE, DK), lambda i: (i, 0)),
            pl.BlockSpec((_BE, D), lambda i: (i, 0)),
        ],
        out_shape=[
            jax.ShapeDtypeStruct((E, DK), jnp.float32),
            jax.ShapeDtypeStruct((E, D), jnp.float32),
        ],
    )(eg, dist, p["Wkg"], p["Wvg"],
      p["W1k"], p["b1k"].reshape(1, RH), p["W2k"], p["b2k"].reshape(1, DK),
      p["W1v"], p["b1v"].reshape(1, RH), p["W2v"], p["b2v"].reshape(1, D))


def _proj(x, wq_ref, wkf_ref, wvf_ref, wvs_ref, q_ref, kf_ref, vf_ref, vs_ref):
    q_ref[...] = jnp.dot(x, wq_ref[...], preferred_element_type=jnp.float32)
    kf_ref[...] = jnp.dot(x, wkf_ref[...], preferred_element_type=jnp.float32)
    vf_ref[...] = jnp.dot(x, wvf_ref[...], preferred_element_type=jnp.float32)
    vs_ref[...] = jnp.dot(x, wvs_ref[...], preferred_element_type=jnp.float32)


def _node_dense_body(x_ref, wq_ref, wkf_ref, wvf_ref, wvs_ref,
                     q_ref, kf_ref, vf_ref, vs_ref):
    _proj(x_ref[...], wq_ref, wkf_ref, wvf_ref, wvs_ref,
          q_ref, kf_ref, vf_ref, vs_ref)


_NODE_OUT = [
    jax.ShapeDtypeStruct((N, D), jnp.float32),
    jax.ShapeDtypeStruct((N, D), jnp.float32),
    jax.ShapeDtypeStruct((N, D), jnp.float32),
    jax.ShapeDtypeStruct((N, D), jnp.float32),
]


def _pad_w(w):
    return jnp.pad(w, ((0, 0), (0, D - w.shape[1])))


def _node_dense(x, p):
    return pl.pallas_call(_node_dense_body, out_shape=_NODE_OUT)(
        x, _pad_w(p["Wq"]), _pad_w(p["Wkf"]), p["Wvf"], p["Wvs"])


def _combine(num_ref, den_ref, vs_ref):
    num = num_ref[0] + num_ref[1] + vs_ref[...]
    den = 1.0 + den_ref[0, :N] + den_ref[1, :N]
    return num / den[:, None]


def _combine_node_body(num_ref, den_ref, vsp_ref, wq_ref, wkf_ref, wvf_ref,
                       wvs_ref, q_ref, kf_ref, vf_ref, vs_ref):
    x = _combine(num_ref, den_ref, vsp_ref)
    _proj(x, wq_ref, wkf_ref, wvf_ref, wvs_ref, q_ref, kf_ref, vf_ref, vs_ref)


def _combine_node(num_p, den_p, vs_prev, p):
    return pl.pallas_call(_combine_node_body, out_shape=_NODE_OUT)(
        num_p, den_p, vs_prev, _pad_w(p["Wq"]), _pad_w(p["Wkf"]),
        p["Wvf"], p["Wvs"])


def _combine_only_body(num_ref, den_ref, vsp_ref, x_ref):
    x_ref[...] = _combine(num_ref, den_ref, vsp_ref)


def _combine_only(num_p, den_p, vs_prev):
    return pl.pallas_call(
        _combine_only_body,
        out_shape=jax.ShapeDtypeStruct((N, D), jnp.float32),
    )(num_p, den_p, vs_prev)


_MESH = plsc.VectorSubcoreMesh(
    core_axis_name="c", subcore_axis_name="s", num_cores=NC, num_subcores=NS)

NDEN = 10112
_DEN0 = 7 * C * DK
_DENR = NDEN - _DEN0


@functools.partial(
    pl.kernel,
    out_type=[
        jax.ShapeDtypeStruct((NC, N, D), jnp.float32),
        jax.ShapeDtypeStruct((NC, NDEN), jnp.float32),
    ],
    mesh=_MESH,
    compiler_params=pltpu.CompilerParams(needs_layout_passes=False),
    scratch_types=[
        pltpu.VMEM((NB * C,), jnp.int32),
        pltpu.VMEM((NB * C,), jnp.int32),
        pltpu.VMEM((C, D), jnp.float32),
        pltpu.VMEM((C, D), jnp.float32),
        pltpu.VMEM((C, D), jnp.float32),
        pltpu.VMEM((C * DK,), jnp.float32),
        pltpu.VMEM((C, D), jnp.float32),
        pltpu.VMEM((C,), jnp.float32),
        pltpu.VMEM((C,), jnp.int32),
        pltpu.VMEM((C,), jnp.int32),
        pltpu.VMEM((C, D), jnp.float32),
        pltpu.VMEM((C, D), jnp.float32),
        pltpu.VMEM((C, D), jnp.float32),
        pltpu.VMEM((C * DK,), jnp.float32),
        pltpu.VMEM((C, D), jnp.float32),
        pltpu.VMEM((C,), jnp.float32),
        pltpu.VMEM((C,), jnp.int32),
        pltpu.VMEM((C,), jnp.int32),
        pltpu.VMEM_SHARED((N, D), jnp.float32),
        pltpu.VMEM_SHARED((NDEN,), jnp.float32),
        pltpu.SemaphoreType.DMA,
        pltpu.SemaphoreType.DMA,
        pltpu.SemaphoreType.DMA,
        pltpu.SemaphoreType.DMA,
    ],
)
def _sc_edge(src_hbm, dst_hbm, q_hbm, kf_hbm, vf_hbm, gk_hbm, gv_hbm,
             num_out, den_out,
             srcb, dstb,
             q0, kf0, vf0, gk0, gv0, ev0, sc0, dc0,
             q1, kf1, vf1, gk1, gv1, ev1, sc1, dc1,
             num_sh, den_sh, dsem0, dsem1, ssem0, ssem1):
    cid = lax.axis_index("c")
    sid = lax.axis_index("s")
    wid = sid * NC + cid
    cbase = wid * CPW
    z16 = jnp.zeros((L,), jnp.float32)
    sets = [(q0, kf0, vf0, gk0, gv0, ev0, sc0, dc0, dsem0, ssem0),
            (q1, kf1, vf1, gk1, gv1, ev1, sc1, dc1, dsem1, ssem1)]

    def _zvf(i, carry):
        for d in range(D // L):
            vf0[i, pl.ds(d * L, L)] = z16
        return carry
    lax.fori_loop(0, C, _zvf, 0)

    def _zgk(i, carry):
        gk0[pl.ds(i * L, L)] = z16
        return carry
    lax.fori_loop(0, C * DK // L, _zgk, 0)

    for k in range(CPW // NS):
        pltpu.sync_copy(vf0, num_sh.at[pl.ds((sid + k * NS) * C, C)])
    @pl.when(sid < CPW - (CPW // NS) * NS)
    def _():
        pltpu.sync_copy(vf0, num_sh.at[pl.ds((sid + (CPW // NS) * NS) * C, C)])
    @pl.when(sid < 7)
    def _():
        pltpu.sync_copy(gk0, den_sh.at[pl.ds(sid * C * DK, C * DK)])
    @pl.when(sid == 7)
    def _():
        pltpu.sync_copy(gk0.at[pl.ds(0, _DENR)], den_sh.at[pl.ds(_DEN0, _DENR)])
    plsc.subcore_barrier()

    def _fetch_idx(batch):
        off = (cbase + batch * NB) * C
        pltpu.sync_copy(src_hbm.at[pl.ds(off, NB * C)], srcb)
        pltpu.sync_copy(dst_hbm.at[pl.ds(off, NB * C)], dstb)

    def _start_inputs(ci, s):
        (qr, kfr, vfr, gkr, gvr, _, scc, dcc, dsem, _) = sets[s]
        gc = cbase + ci
        base = lax.rem(ci, NB) * C
        for off in (0, 16, 24):
            scc[pl.ds(off, L)] = srcb[pl.ds(base + off, L)]
            dcc[pl.ds(off, L)] = dstb[pl.ds(base + off, L)]
        pltpu.async_copy(q_hbm.at[dcc], qr, dsem)
        pltpu.async_copy(kf_hbm.at[scc], kfr, dsem)
        pltpu.async_copy(vf_hbm.at[scc], vfr, dsem)
        pltpu.async_copy(gk_hbm.at[gc], gkr, dsem)
        pltpu.async_copy(gv_hbm.at[pl.ds(gc * C, C)], gvr, dsem)

    def _wait_inputs(s):
        (qr, kfr, vfr, gkr, gvr, _, scc, dcc, dsem, _) = sets[s]
        pltpu.make_async_copy(q_hbm.at[dcc], qr, dsem).wait()
        pltpu.make_async_copy(kf_hbm.at[scc], kfr, dsem).wait()
        pltpu.make_async_copy(vf_hbm.at[scc], vfr, dsem).wait()
        pltpu.make_async_copy(gk_hbm.at[0], gkr, dsem).wait()
        pltpu.make_async_copy(gv_hbm.at[pl.ds(0, C)], gvr, dsem).wait()

    def _wait_scatters(s):
        (_, _, vfr, _, _, evr, _, dcc, _, ssem) = sets[s]
        pltpu.make_async_copy(vfr, num_sh.at[dcc], ssem).wait()
        pltpu.make_async_copy(evr, den_sh.at[dcc], ssem).wait()

    def _compute_and_scatter(s):
        (qr, kfr, vfr, gkr, gvr, evr, _, dcc, _, ssem) = sets[s]
        @plsc.parallel_loop(0, 3, step=1, unroll=1)
        def _logits(i):
            g0 = jnp.minimum(i * L, C - L)
            acc = z16
            eidx = lax.iota(jnp.int32, L) + g0
            for j in range(DK):
                jidx = jnp.full((L,), j, jnp.int32)
                acc = acc + (plsc.load_gather(qr, [eidx, jidx]) *
                             plsc.load_gather(kfr, [eidx, jidx]) *
                             gkr[pl.ds(j * C + g0, L)])
            evr[pl.ds(g0, L)] = jnp.exp(acc * INV_SQRT_DK - 1.0)
        @plsc.parallel_loop(0, C, step=1, unroll=4)
        def _vrow(e):
            evv = plsc.load_gather(evr, [jnp.full((L,), e, jnp.int32)])
            for d in range(D // L):
                sl = pl.ds(d * L, L)
                vfr[e, sl] = evv * vfr[e, sl] * gvr[e, sl]
        pltpu.async_copy(vfr, num_sh.at[dcc], ssem, add=True)
        pltpu.async_copy(evr, den_sh.at[dcc], ssem, add=True)

    _fetch_idx(0)
    _start_inputs(0, 0)

    def _body(k, carry):
        a = 2 * k
        @pl.when(k > 0)
        def _():
            _wait_scatters(1)
        _start_inputs(a + 1, 1)
        _wait_inputs(0)
        _compute_and_scatter(0)
        _wait_inputs(1)
        @pl.when(k < HB - 1)
        def _():
            @pl.when(lax.rem(a + 2, NB) == 0)
            def _():
                _fetch_idx((a + 2) // NB)
            _wait_scatters(0)
            _start_inputs(a + 2, 0)
        _compute_and_scatter(1)
        return carry

    lax.fori_loop(0, HB, _body, 0)
    _wait_scatters(0)
    _wait_scatters(1)

    plsc.subcore_barrier()
    for k in range(CPW // NS):
        r0 = (sid + k * NS) * C
        pltpu.sync_copy(num_sh.at[pl.ds(r0, C)], vf0)
        pltpu.sync_copy(vf0, num_out.at[cid, pl.ds(r0, C)])
    @pl.when(sid < CPW - (CPW // NS) * NS)
    def _():
        r0 = (sid + (CPW // NS) * NS) * C
        pltpu.sync_copy(num_sh.at[pl.ds(r0, C)], vf0)
        pltpu.sync_copy(vf0, num_out.at[cid, pl.ds(r0, C)])
    @pl.when(sid < 7)
    def _():
        pltpu.sync_copy(den_sh.at[pl.ds(sid * C * DK, C * DK)], gk0)
        pltpu.sync_copy(gk0, den_out.at[cid, pl.ds(sid * C * DK, C * DK)])
    @pl.when(sid == 7)
    def _():
        pltpu.sync_copy(den_sh.at[pl.ds(_DEN0, _DENR)], gk0.at[pl.ds(0, _DENR)])
        pltpu.sync_copy(gk0.at[pl.ds(0, _DENR)],
                        den_out.at[cid, pl.ds(_DEN0, _DENR)])


def kernel(edge_index, node_features, edge_features, distances, params):
    src = edge_index[0].astype(jnp.int32)
    dst = edge_index[1].astype(jnp.int32)
    p1, p2 = params

    gk1, gv1 = _edge_dense(edge_features, distances, p1)
    gk2, gv2 = _edge_dense(edge_features, distances, p2)
    gk1 = gk1.reshape(E // C, C, DK).transpose(0, 2, 1).reshape(E // C, DK * C)
    gk2 = gk2.reshape(E // C, C, DK).transpose(0, 2, 1).reshape(E // C, DK * C)

    q1, kf1, vf1, vs1 = _node_dense(node_features, p1)
    num1, den1 = _sc_edge(src, dst, q1, kf1, vf1, gk1, gv1)
    q2, kf2, vf2, vs2 = _combine_node(num1, den1, vs1, p2)
    num2, den2 = _sc_edge(src, dst, q2, kf2, vf2, gk2, gv2)
    return _combine_only(num2, den2, vs2)
